# baseline (speedup 1.0000x reference)
import functools

import jax
import jax.numpy as jnp
from jax import lax
from jax.experimental import pallas as pl
from jax.experimental.pallas import tpu as pltpu
from jax.experimental.pallas import tpu_sc as plsc

BATCH = 16384
EMB = 64
CAT_EMB = 32

_NC = 2
_NS = 16
_NW = _NC * _NS
_CHUNK = 128
_B_PER_W = BATCH // _NW
_NCH = _B_PER_W // _CHUNK

_sc_mesh = plsc.VectorSubcoreMesh(core_axis_name="c", subcore_axis_name="s")


@functools.partial(
    pl.kernel,
    out_type=[
        jax.ShapeDtypeStruct((BATCH, EMB), jnp.float32),
        jax.ShapeDtypeStruct((BATCH, CAT_EMB), jnp.float32),
    ],
    mesh=_sc_mesh,
    scratch_types=[
        pltpu.VMEM((_NCH, _CHUNK), jnp.int32),
        pltpu.VMEM((_NCH, _CHUNK), jnp.int32),
        pltpu.VMEM((_B_PER_W, EMB), jnp.float32),
        pltpu.VMEM((_B_PER_W, CAT_EMB), jnp.float32),
        pltpu.SemaphoreType.DMA,
    ],
    compiler_params=pltpu.CompilerParams(use_tc_tiling_on_sc=False),
)
def _sc_gather(ids_hbm, cids_hbm, item_table_hbm, cat_table_hbm,
               item_out, cat_out, idx_v, cidx_v, rows_v, crows_v, sem):
    wid = lax.axis_index("s") * _NC + lax.axis_index("c")
    base = wid * _B_PER_W
    pltpu.sync_copy(ids_hbm.at[wid], idx_v)
    pltpu.sync_copy(cids_hbm.at[wid], cidx_v)
    copies = []
    for k in range(_NCH):
        copies.append(pltpu.async_copy(
            item_table_hbm.at[idx_v.at[k]],
            rows_v.at[pl.ds(k * _CHUNK, _CHUNK)], sem))
        copies.append(pltpu.async_copy(
            cat_table_hbm.at[cidx_v.at[k]],
            crows_v.at[pl.ds(k * _CHUNK, _CHUNK)], sem))
    for cp in copies:
        cp.wait()
    pltpu.sync_copy(rows_v, item_out.at[pl.ds(base, _B_PER_W)])
    pltpu.sync_copy(crows_v, cat_out.at[pl.ds(base, _B_PER_W)])


_BM = 2048


def _dense_body(x1_ref, x2_ref, w1_ref, w2_ref, b_ref, o_ref):
    o_ref[...] = (
        jnp.dot(x1_ref[...], w1_ref[...], preferred_element_type=jnp.float32)
        + jnp.dot(x2_ref[...], w2_ref[...], preferred_element_type=jnp.float32)
        + b_ref[...]
    )


_tc_dense = pl.pallas_call(
    _dense_body,
    grid=(BATCH // _BM,),
    in_specs=[
        pl.BlockSpec((_BM, EMB), lambda i: (i, 0)),
        pl.BlockSpec((_BM, CAT_EMB), lambda i: (i, 0)),
        pl.BlockSpec((EMB, EMB), lambda i: (0, 0)),
        pl.BlockSpec((CAT_EMB, EMB), lambda i: (0, 0)),
        pl.BlockSpec((1, EMB), lambda i: (0, 0)),
    ],
    out_specs=pl.BlockSpec((_BM, EMB), lambda i: (i, 0)),
    out_shape=jax.ShapeDtypeStruct((BATCH, EMB), jnp.float32),
)


def kernel(item_id, item_category, item_table, cat_table, W, b):
    ids = item_id.reshape(_NW, _NCH, _CHUNK)
    cids = item_category.reshape(_NW, _NCH, _CHUNK)
    item_emb, cat_emb = _sc_gather(ids, cids, item_table, cat_table)
    return _tc_dense(item_emb, cat_emb, W[:EMB], W[EMB:], b.reshape(1, EMB))
